# baseline (speedup 1.0000x reference)
import functools
import math

import numpy as np

import jax
import jax.numpy as jnp
from jax.experimental import pallas as pl
from jax.experimental.pallas import tpu as pltpu


def _interp_arrays(out_size, in_size):
    if out_size == 1:
        src = np.zeros((1,), np.float32)
    else:
        src = np.arange(out_size, dtype=np.float32) * np.float32(
            (in_size - 1) / (out_size - 1)
        )
    i0 = np.clip(np.floor(src).astype(np.int32), 0, in_size - 1)
    i1 = np.minimum(i0 + 1, in_size - 1)
    w1 = src - i0.astype(np.float32)
    w0 = np.float32(1.0) - w1
    return i0, i1, w0, w1


def _interp_matrix(out_size, in_size):
    i0, i1, w0, w1 = _interp_arrays(out_size, in_size)
    m = np.zeros((out_size, in_size), np.float32)
    m[np.arange(out_size), i0] += w0
    m[np.arange(out_size), i1] += w1
    return m


def _resize_kernel(x_hbm, wwt_ref, wh_ref, o_hbm,
                   x_buf, o_buf, in_sem, out_sem, *, chunk):
    core = pl.program_id(0)
    nsl = x_hbm.shape[0] // pl.num_programs(0)
    base = core * nsl
    nch = nsl // chunk

    wwt = wwt_ref[...].astype(jnp.bfloat16)
    wh = wh_ref[...].astype(jnp.bfloat16)
    h = x_hbm.shape[1]
    wo = wwt.shape[1]

    def dma_in(slot, step):
        pltpu.make_async_copy(x_hbm.at[pl.ds(base + step * chunk, chunk)],
                              x_buf.at[slot], in_sem.at[slot]).start()

    def wait_in(slot):
        pltpu.make_async_copy(x_hbm.at[pl.ds(base, chunk)],
                              x_buf.at[slot], in_sem.at[slot]).wait()

    def dma_out(slot, step):
        pltpu.make_async_copy(o_buf.at[slot],
                              o_hbm.at[pl.ds(base + step * chunk, chunk)],
                              out_sem.at[slot]).start()

    def wait_out(slot):
        pltpu.make_async_copy(o_buf.at[slot],
                              o_hbm.at[pl.ds(base, chunk)],
                              out_sem.at[slot]).wait()

    dma_in(0, 0)
    for step in range(nch):
        cur, nxt = step % 2, (step + 1) % 2
        if step + 1 < nch:
            dma_in(nxt, step + 1)
        wait_in(cur)
        if step >= 2:
            wait_out(cur)
        x = x_buf[cur].astype(jnp.bfloat16)
        u = jnp.dot(x.reshape(chunk * h, x.shape[2]), wwt,
                    preferred_element_type=jnp.float32).reshape(chunk, h, wo)
        for b in range(chunk):
            o_buf[cur, b] = jnp.dot(wh, u[b].astype(jnp.bfloat16),
                                    preferred_element_type=jnp.float32)
        dma_out(cur, step)
    for k in range(max(0, nch - 2), nch):
        wait_out(k % 2)


def kernel(x):
    vel_resize = 2.0
    factor = 1.0 / vel_resize
    N, C, H, W = x.shape
    H_out = int(math.floor(H * factor))
    W_out = int(math.floor(W * factor))
    assert H == 2 * H_out and W == 2 * W_out
    B = N * C
    assert B % 2 == 0

    wwt = jnp.asarray(np.ascontiguousarray(_interp_matrix(W_out, W).T))
    wh = jnp.asarray(np.float32(factor) * _interp_matrix(H_out, H))

    nsl = B // 2
    chunk = 8
    while chunk > 1 and nsl % chunk:
        chunk //= 2

    body = functools.partial(_resize_kernel, chunk=chunk)
    out3 = pl.pallas_call(
        body,
        out_shape=jax.ShapeDtypeStruct((B, H_out, W_out), x.dtype),
        grid=(2,),
        in_specs=[
            pl.BlockSpec(memory_space=pl.ANY),
            pl.BlockSpec((W, W_out), lambda b: (0, 0)),
            pl.BlockSpec((H_out, H), lambda b: (0, 0)),
        ],
        out_specs=pl.BlockSpec(memory_space=pl.ANY),
        scratch_shapes=[
            pltpu.VMEM((2, chunk, H, W), jnp.float32),
            pltpu.VMEM((2, chunk, H_out, W_out), jnp.float32),
            pltpu.SemaphoreType.DMA((2,)),
            pltpu.SemaphoreType.DMA((2,)),
        ],
        compiler_params=pltpu.CompilerParams(
            dimension_semantics=("parallel",),
            vmem_limit_bytes=int(64 * 1024 * 1024 * 0.85),
        ),
    )(x.reshape(B, H, W), wwt, wh)
    return out3.reshape(N, C, H_out, W_out)


# speedup vs baseline: 1.1576x; 1.1576x over previous
import functools
import math

import numpy as np

import jax
import jax.numpy as jnp
from jax.experimental import pallas as pl
from jax.experimental.pallas import tpu as pltpu


def _interp_arrays(out_size, in_size):
    if out_size == 1:
        src = np.zeros((1,), np.float32)
    else:
        src = np.arange(out_size, dtype=np.float32) * np.float32(
            (in_size - 1) / (out_size - 1)
        )
    i0 = np.clip(np.floor(src).astype(np.int32), 0, in_size - 1)
    i1 = np.minimum(i0 + 1, in_size - 1)
    w1 = src - i0.astype(np.float32)
    w0 = np.float32(1.0) - w1
    return i0, i1, w0, w1


def _interp_matrix(out_size, in_size):
    i0, i1, w0, w1 = _interp_arrays(out_size, in_size)
    m = np.zeros((out_size, in_size), np.float32)
    m[np.arange(out_size), i0] += w0
    m[np.arange(out_size), i1] += w1
    return m


def _resize_kernel(x_hbm, wwt_ref, wh_ref, o_hbm,
                   x_buf, o_buf, in_sem, out_sem, *, chunk):
    core = pl.program_id(0)
    nsl = x_hbm.shape[0] // pl.num_programs(0)
    base = core * nsl
    nch = nsl // chunk

    wwt = wwt_ref[...].astype(jnp.bfloat16)
    wh = wh_ref[...].astype(jnp.bfloat16)
    h = x_hbm.shape[1]
    wo = wwt.shape[1]

    def dma_in(slot, step):
        pltpu.make_async_copy(x_hbm.at[pl.ds(base + step * chunk, chunk)],
                              x_buf.at[slot], in_sem.at[slot]).start()

    def wait_in(slot):
        pltpu.make_async_copy(x_hbm.at[pl.ds(base, chunk)],
                              x_buf.at[slot], in_sem.at[slot]).wait()

    def dma_out(slot, step):
        pltpu.make_async_copy(o_buf.at[slot],
                              o_hbm.at[pl.ds(base + step * chunk, chunk)],
                              out_sem.at[slot]).start()

    def wait_out(slot):
        pltpu.make_async_copy(o_buf.at[slot],
                              o_hbm.at[pl.ds(base, chunk)],
                              out_sem.at[slot]).wait()

    dma_in(0, 0)
    for step in range(nch):
        cur, nxt = step % 2, (step + 1) % 2
        if step + 1 < nch:
            dma_in(nxt, step + 1)
        wait_in(cur)
        if step >= 2:
            wait_out(cur)
        x = x_buf[cur].astype(jnp.bfloat16)
        u = jnp.dot(x.reshape(chunk * h, x.shape[2]), wwt,
                    preferred_element_type=jnp.float32).reshape(chunk, h, wo)
        for b in range(chunk):
            o_buf[cur, b] = jnp.dot(wh, u[b].astype(jnp.bfloat16),
                                    preferred_element_type=jnp.float32)
        dma_out(cur, step)
    for k in range(max(0, nch - 2), nch):
        wait_out(k % 2)


def kernel(x):
    vel_resize = 2.0
    factor = 1.0 / vel_resize
    N, C, H, W = x.shape
    H_out = int(math.floor(H * factor))
    W_out = int(math.floor(W * factor))
    assert H == 2 * H_out and W == 2 * W_out
    B = N * C
    assert B % 2 == 0

    wwt = jnp.asarray(np.ascontiguousarray(_interp_matrix(W_out, W).T))
    wh = jnp.asarray(np.float32(factor) * _interp_matrix(H_out, H))

    nsl = B // 2
    chunk = 16
    while chunk > 1 and nsl % chunk:
        chunk //= 2

    body = functools.partial(_resize_kernel, chunk=chunk)
    out3 = pl.pallas_call(
        body,
        out_shape=jax.ShapeDtypeStruct((B, H_out, W_out), x.dtype),
        grid=(2,),
        in_specs=[
            pl.BlockSpec(memory_space=pl.ANY),
            pl.BlockSpec((W, W_out), lambda b: (0, 0)),
            pl.BlockSpec((H_out, H), lambda b: (0, 0)),
        ],
        out_specs=pl.BlockSpec(memory_space=pl.ANY),
        scratch_shapes=[
            pltpu.VMEM((2, chunk, H, W), jnp.float32),
            pltpu.VMEM((2, chunk, H_out, W_out), jnp.float32),
            pltpu.SemaphoreType.DMA((2,)),
            pltpu.SemaphoreType.DMA((2,)),
        ],
        compiler_params=pltpu.CompilerParams(
            dimension_semantics=("parallel",),
            vmem_limit_bytes=int(64 * 1024 * 1024 * 0.85),
        ),
    )(x.reshape(B, H, W), wwt, wh)
    return out3.reshape(N, C, H_out, W_out)


# speedup vs baseline: 1.3270x; 1.1464x over previous
import math

import numpy as np

import jax
import jax.numpy as jnp
from jax.experimental import pallas as pl
from jax.experimental.pallas import tpu as pltpu


def _interp_arrays(out_size, in_size):
    if out_size == 1:
        src = np.zeros((1,), np.float32)
    else:
        src = np.arange(out_size, dtype=np.float32) * np.float32(
            (in_size - 1) / (out_size - 1)
        )
    i0 = np.clip(np.floor(src).astype(np.int32), 0, in_size - 1)
    i1 = np.minimum(i0 + 1, in_size - 1)
    w1 = src - i0.astype(np.float32)
    w0 = np.float32(1.0) - w1
    return i0, i1, w0, w1


def _interp_matrix(out_size, in_size):
    i0, i1, w0, w1 = _interp_arrays(out_size, in_size)
    m = np.zeros((out_size, in_size), np.float32)
    m[np.arange(out_size), i0] += w0
    m[np.arange(out_size), i1] += w1
    return m


def _resize_kernel(x_ref, wwt_ref, wh_ref, o_ref):
    wwt = wwt_ref[...].astype(jnp.bfloat16)
    wh = wh_ref[...].astype(jnp.bfloat16)
    wo = wwt.shape[1]
    x = x_ref[...].astype(jnp.bfloat16)
    tb, h, w = x.shape
    u = jnp.dot(x.reshape(tb * h, w), wwt,
                preferred_element_type=jnp.float32).reshape(tb, h, wo)
    for b in range(tb):
        o_ref[b] = jnp.dot(wh, u[b].astype(jnp.bfloat16),
                           preferred_element_type=jnp.float32)


def kernel(x):
    vel_resize = 2.0
    factor = 1.0 / vel_resize
    N, C, H, W = x.shape
    H_out = int(math.floor(H * factor))
    W_out = int(math.floor(W * factor))
    assert H == 2 * H_out and W == 2 * W_out
    B = N * C

    wwt = jnp.asarray(np.ascontiguousarray(_interp_matrix(W_out, W).T))
    wh = jnp.asarray(np.float32(factor) * _interp_matrix(H_out, H))

    CORES, TB = 2, 16
    while TB > 1 and B % (CORES * TB):
        TB //= 2
    if B % (CORES * TB):
        CORES, TB = 1, 1
    nch = B // (CORES * TB)

    out3 = pl.pallas_call(
        _resize_kernel,
        out_shape=jax.ShapeDtypeStruct((B, H_out, W_out), x.dtype),
        grid=(CORES, nch),
        in_specs=[
            pl.BlockSpec((TB, H, W), lambda c, k: (c * nch + k, 0, 0)),
            pl.BlockSpec((W, W_out), lambda c, k: (0, 0)),
            pl.BlockSpec((H_out, H), lambda c, k: (0, 0)),
        ],
        out_specs=pl.BlockSpec((TB, H_out, W_out),
                               lambda c, k: (c * nch + k, 0, 0)),
        compiler_params=pltpu.CompilerParams(
            dimension_semantics=("parallel", "arbitrary"),
            vmem_limit_bytes=int(64 * 1024 * 1024 * 0.85),
        ),
    )(x.reshape(B, H, W), wwt, wh)
    return out3.reshape(N, C, H_out, W_out)
